# baseline (speedup 1.0000x reference)
import functools

import jax
import jax.numpy as jnp
from jax import lax
from jax.experimental import pallas as pl
from jax.experimental.pallas import tpu as pltpu
from jax.experimental.pallas import tpu_sc as plsc

_B = 4096
_N = 200
_D = 128
_HALF = _D // 2
_TOK = _B * _N
_NW = 32
_PER_W = _TOK // _NW
_CH = 128
_NB = 5
_NCH = _PER_W // _CH
_NROW = 3001

_mesh = plsc.VectorSubcoreMesh(core_axis_name="c", subcore_axis_name="s")


@functools.partial(
    pl.kernel,
    out_type=jax.ShapeDtypeStruct((2 * _TOK, _HALF), jnp.float32),
    mesh=_mesh,
    compiler_params=pltpu.CompilerParams(
        use_tc_tiling_on_sc=False, needs_layout_passes=False),
    scratch_types=[
        pltpu.VMEM((_NB, _CH), jnp.int32),
        pltpu.VMEM((_NB, _CH), jnp.int32),
        pltpu.VMEM((_NB, 2 * _CH), jnp.int32),
        pltpu.VMEM((_NB, 2 * _CH, _HALF), jnp.float32),
        pltpu.VMEM_SHARED((_NROW + 3, _HALF), jnp.float32),
        [pltpu.SemaphoreType.DMA] * _NB,
        [pltpu.SemaphoreType.DMA] * _NB,
        [pltpu.SemaphoreType.DMA] * _NB,
    ],
)
def _pe_gather(doy_hbm, grp_hbm, tab_hbm, out_hbm,
               doy_v, grp_v, idx_v, rows_v, tab_sh, gsems, wsems, isems):
    wid = lax.axis_index("s") * 2 + lax.axis_index("c")
    base0 = wid * _PER_W
    lane = jax.lax.iota(jnp.int32, 16)

    @pl.when(lax.axis_index("s") == 0)
    def _stage():
        pltpu.sync_copy(tab_hbm, tab_sh)

    plsc.subcore_barrier()

    def start_idx_loads(c, b):
        base = base0 + c * _CH
        pltpu.async_copy(doy_hbm.at[pl.ds(base, _CH)], doy_v.at[b], isems[b])
        pltpu.async_copy(grp_hbm.at[pl.ds(base, _CH)], grp_v.at[b], isems[b])

    def drain(src, dst, sem):
        pltpu.make_async_copy(src, dst, sem).wait()

    for b in range(_NB):
        start_idx_loads(b, b)

    def body(i, _):
        gs = [None] * _NB
        for b in range(_NB):
            drain(doy_hbm.at[pl.ds(0, _CH)], doy_v.at[b], isems[b])
            drain(grp_hbm.at[pl.ds(0, _CH)], grp_v.at[b], isems[b])
            for j in range(_CH // 16):
                dv = doy_v[b, pl.ds(j * 16, 16)]
                gv = grp_v[b, pl.ds(j * 16, 16)] + _NROW
                pos = (j * 16) * 2 + lane * 2
                plsc.store_scatter(idx_v.at[b], [pos], dv)
                plsc.store_scatter(idx_v.at[b], [pos + 1], gv)
            @pl.when(i > 0)
            def _():
                drain(out_hbm.at[pl.ds(0, 2 * _CH)], rows_v.at[b], wsems[b])

            gs[b] = pltpu.async_copy(
                tab_sh.at[idx_v.at[b]], rows_v.at[b], gsems[b])
        @pl.when(i < _NCH // _NB - 1)
        def _():
            for b in range(_NB):
                start_idx_loads((i + 1) * _NB + b, b)

        for b in range(_NB):
            base = base0 + (i * _NB + b) * _CH
            gs[b].wait()
            pltpu.async_copy(
                rows_v.at[b], out_hbm.at[pl.ds(2 * base, 2 * _CH)], wsems[b])
        return 0

    lax.fori_loop(0, _NCH // _NB, body, 0)
    for b in range(_NB):
        drain(out_hbm.at[pl.ds(0, 2 * _CH)], rows_v.at[b], wsems[b])


def kernel(doy, group, pe_doy, pe_group):
    doy_f = doy.reshape(_TOK).astype(jnp.int32)
    grp_f = group.reshape(_TOK).astype(jnp.int32)
    tab = jnp.concatenate([pe_doy, pe_group], axis=0)
    out = _pe_gather(doy_f, grp_f, tab)
    return out.reshape(_B, _N, _D)[:, :, :, None, None]


# speedup vs baseline: 1.0117x; 1.0117x over previous
import functools

import jax
import jax.numpy as jnp
from jax import lax
from jax.experimental import pallas as pl
from jax.experimental.pallas import tpu as pltpu
from jax.experimental.pallas import tpu_sc as plsc

_B = 4096
_N = 200
_D = 128
_HALF = _D // 2
_TOK = _B * _N
_NW = 32
_PER_W = _TOK // _NW
_CH = 128
_NB = 4
_NCH = _PER_W // _CH
_NROW = 3001

_mesh = plsc.VectorSubcoreMesh(core_axis_name="c", subcore_axis_name="s")


@functools.partial(
    pl.kernel,
    out_type=jax.ShapeDtypeStruct((2 * _TOK, _HALF), jnp.float32),
    mesh=_mesh,
    compiler_params=pltpu.CompilerParams(
        use_tc_tiling_on_sc=False, needs_layout_passes=False),
    scratch_types=[
        pltpu.VMEM((_PER_W,), jnp.int32),
        pltpu.VMEM((_PER_W,), jnp.int32),
        pltpu.VMEM((_NB, 2 * _CH), jnp.int32),
        pltpu.VMEM((_NB, 2 * _CH, _HALF), jnp.float32),
        pltpu.VMEM_SHARED((_NROW + 3, _HALF), jnp.float32),
        pltpu.SemaphoreType.DMA,
        [pltpu.SemaphoreType.DMA] * _NB,
        [pltpu.SemaphoreType.DMA] * _NB,
    ],
)
def _pe_gather(doy_hbm, grp_hbm, tab_hbm, out_hbm,
               doy_all, grp_all, idx_v, rows_v, tab_sh, isem, gsems, wsems):
    wid = lax.axis_index("s") * 2 + lax.axis_index("c")
    base0 = wid * _PER_W
    lane = jax.lax.iota(jnp.int32, 16)

    pltpu.async_copy(doy_hbm.at[pl.ds(base0, _PER_W)], doy_all, isem)
    pltpu.async_copy(grp_hbm.at[pl.ds(base0, _PER_W)], grp_all, isem)

    @pl.when(lax.axis_index("s") == 0)
    def _stage():
        pltpu.sync_copy(tab_hbm, tab_sh)

    plsc.subcore_barrier()
    pltpu.make_async_copy(doy_hbm.at[pl.ds(0, _PER_W)], doy_all, isem).wait()
    pltpu.make_async_copy(grp_hbm.at[pl.ds(0, _PER_W)], grp_all, isem).wait()

    def drain(src, dst, sem):
        pltpu.make_async_copy(src, dst, sem).wait()

    def body(i, _):
        gs = [None] * _NB
        for b in range(_NB):
            off = (i * _NB + b) * _CH
            for j in range(_CH // 16):
                dv = doy_all[pl.ds(off + j * 16, 16)]
                gv = grp_all[pl.ds(off + j * 16, 16)] + _NROW
                pos = (j * 16) * 2 + lane * 2
                plsc.store_scatter(idx_v.at[b], [pos], dv)
                plsc.store_scatter(idx_v.at[b], [pos + 1], gv)
            @pl.when(i > 0)
            def _():
                drain(out_hbm.at[pl.ds(0, 2 * _CH)], rows_v.at[b], wsems[b])

            gs[b] = pltpu.async_copy(
                tab_sh.at[idx_v.at[b]], rows_v.at[b], gsems[b])

        for b in range(_NB):
            base = base0 + (i * _NB + b) * _CH
            gs[b].wait()
            pltpu.async_copy(
                rows_v.at[b], out_hbm.at[pl.ds(2 * base, 2 * _CH)], wsems[b])
        return 0

    lax.fori_loop(0, _NCH // _NB, body, 0)
    for b in range(_NB):
        drain(out_hbm.at[pl.ds(0, 2 * _CH)], rows_v.at[b], wsems[b])


def kernel(doy, group, pe_doy, pe_group):
    doy_f = doy.reshape(_TOK).astype(jnp.int32)
    grp_f = group.reshape(_TOK).astype(jnp.int32)
    tab = jnp.concatenate([pe_doy, pe_group], axis=0)
    out = _pe_gather(doy_f, grp_f, tab)
    return out.reshape(_B, _N, _D)[:, :, :, None, None]
